# Initial kernel scaffold; baseline (speedup 1.0000x reference)
import jax
import jax.numpy as jnp
from jax.experimental import pallas as pl


def kernel(x):
    raise NotImplementedError("write your pallas kernel here")


# speedup vs baseline: 2.0634x; 2.0634x over previous
import jax
import jax.numpy as jnp
from jax import lax
from jax.experimental import pallas as pl

_N = 4096
_K = 2048
_LOG2N = 12
_SUB = 3


def _pbit(j):
    hi = _LOG2N - _SUB
    return j + _SUB if j < hi else j - hi


def _sort_block(x_ref, o_ref):
    T = x_ref[...]
    C = T.shape[1]
    sidx = lax.broadcasted_iota(jnp.int32, (_N, 1), 0)
    for st in range(1, _LOG2N + 1):
        if st == _LOG2N:
            m_full = None
        else:
            m_full = ((sidx >> _pbit(st)) & 1) == 0
        for j in range(st - 1, -1, -1):
            D = 1 << _pbit(j)
            if D >= 8:
                G = _N // (2 * D)
                Tr = T.reshape(G, 2, D, C)
                a = Tr[:, 0]
                b = Tr[:, 1]
                mn = jnp.minimum(a, b)
                mx = jnp.maximum(a, b)
                if m_full is None:
                    na, nb = mx, mn
                else:
                    ma = m_full.reshape(G, 2, D, 1)[:, 0]
                    na = jnp.where(ma, mx, mn)
                    nb = jnp.where(ma, mn, mx)
                T = jnp.concatenate([na[:, None], nb[:, None]], axis=1)
                T = T.reshape(_N, C)
            else:
                up = jnp.concatenate([T[D:], T[:D]], axis=0)
                dn = jnp.concatenate([T[-D:], T[:-D]], axis=0)
                is_a = (sidx & D) == 0
                partner = jnp.where(is_a, up, dn)
                mn = jnp.minimum(T, partner)
                mx = jnp.maximum(T, partner)
                cond = is_a if m_full is None else m_full == is_a
                T = jnp.where(cond, mx, mn)
    o_ref[...] = T.reshape(_N // 8, 8, C)[:, 0:4, :]


def kernel(x):
    B, CH, N = x.shape
    R = B * CH
    xr = x.reshape(R, 8, 512).transpose(2, 1, 0).reshape(_N, R)
    C = 128
    out = pl.pallas_call(
        _sort_block,
        grid=(R // C,),
        in_specs=[pl.BlockSpec((_N, C), lambda g: (0, g))],
        out_specs=pl.BlockSpec((_N // 8, 4, C), lambda g: (0, 0, g)),
        out_shape=jax.ShapeDtypeStruct((_N // 8, 4, R), jnp.float32),
    )(xr)
    y = out.transpose(2, 1, 0).reshape(R, _K)
    return y.reshape(B, CH, _K)


# speedup vs baseline: 3.4758x; 1.6845x over previous
import jax
import jax.numpy as jnp
from jax import lax
from jax.experimental import pallas as pl

_N = 4096
_K = 2048


def _sort_block(x_ref, o_ref):
    T = x_ref[...]
    C = T.shape[1]
    sidx = lax.broadcasted_iota(jnp.int32, (_N, 1), 0)

    for st in range(1, 9):
        for j in range(st - 1, -1, -1):
            Dj = 1 << (j + 3)
            A = _N >> (st + 4)
            Bm = 1 << (st - j - 1)
            Tr = T.reshape(A, 2, Bm, 2, Dj, C)
            d = Tr[:, 0]
            e = Tr[:, 1]
            nda = jnp.maximum(d[:, :, 0], d[:, :, 1])
            ndb = jnp.minimum(d[:, :, 0], d[:, :, 1])
            nea = jnp.minimum(e[:, :, 0], e[:, :, 1])
            neb = jnp.maximum(e[:, :, 0], e[:, :, 1])
            nd = jnp.concatenate([nda[:, :, None], ndb[:, :, None]], axis=2)
            ne = jnp.concatenate([nea[:, :, None], neb[:, :, None]], axis=2)
            T = jnp.concatenate([nd[:, None], ne[:, None]], axis=1)
            T = T.reshape(_N, C)

    for st in range(9, 12):
        m_full = ((sidx >> (st - 9)) & 1) == 0
        for j in range(st - 1, -1, -1):
            if j <= 8:
                Dj = 1 << (j + 3)
                G = _N // (2 * Dj)
                Tr = T.reshape(G, 2, Dj, C)
                a = Tr[:, 0]
                b = Tr[:, 1]
                mn = jnp.minimum(a, b)
                mx = jnp.maximum(a, b)
                ma = m_full.reshape(G, 2, Dj, 1)[:, 0]
                na = jnp.where(ma, mx, mn)
                nb = jnp.where(ma, mn, mx)
                T = jnp.concatenate([na[:, None], nb[:, None]], axis=1)
                T = T.reshape(_N, C)
            else:
                Dj = 1 << (j - 9)
                up = jnp.concatenate([T[Dj:], T[:Dj]], axis=0)
                dn = jnp.concatenate([T[-Dj:], T[:-Dj]], axis=0)
                is_a = (sidx & Dj) == 0
                partner = jnp.where(is_a, up, dn)
                mn = jnp.minimum(T, partner)
                mx = jnp.maximum(T, partner)
                T = jnp.where(m_full == is_a, mx, mn)

    Tr = T.reshape(512, 2, 4, C)
    T2 = jnp.maximum(Tr[:, 0], Tr[:, 1]).reshape(_K, C)
    s2idx = lax.broadcasted_iota(jnp.int32, (_K, 1), 0)
    for j in range(10, -1, -1):
        Dp = (1 << (j + 2)) if j <= 8 else (1 << (j - 9))
        if Dp >= 8:
            G = _K // (2 * Dp)
            Tr2 = T2.reshape(G, 2, Dp, C)
            a = Tr2[:, 0]
            b = Tr2[:, 1]
            na = jnp.maximum(a, b)
            nb = jnp.minimum(a, b)
            T2 = jnp.concatenate([na[:, None], nb[:, None]], axis=1)
            T2 = T2.reshape(_K, C)
        else:
            up = jnp.concatenate([T2[Dp:], T2[:Dp]], axis=0)
            dn = jnp.concatenate([T2[-Dp:], T2[:-Dp]], axis=0)
            is_a = (s2idx & Dp) == 0
            partner = jnp.where(is_a, up, dn)
            mn = jnp.minimum(T2, partner)
            mx = jnp.maximum(T2, partner)
            T2 = jnp.where(is_a, mx, mn)

    o_ref[...] = T2.reshape(512, 4, C)


def kernel(x):
    B, CH, N = x.shape
    R = B * CH
    xr = x.reshape(R, 8, 512).transpose(2, 1, 0).reshape(_N, R)
    C = 128
    out = pl.pallas_call(
        _sort_block,
        grid=(R // C,),
        in_specs=[pl.BlockSpec((_N, C), lambda g: (0, g))],
        out_specs=pl.BlockSpec((_N // 8, 4, C), lambda g: (0, 0, g)),
        out_shape=jax.ShapeDtypeStruct((_N // 8, 4, R), jnp.float32),
    )(xr)
    y = out.transpose(2, 1, 0).reshape(R, _K)
    return y.reshape(B, CH, _K)


# speedup vs baseline: 3.7305x; 1.0733x over previous
import jax
import jax.numpy as jnp
from jax import lax
from jax.experimental import pallas as pl

_N = 4096
_K = 2048


def _pbit(b):
    return b + 3 if b < 9 else b - 9


def _sort_block(x_ref, o_ref):
    T = x_ref[...]
    C = T.shape[1]
    sidx = lax.broadcasted_iota(jnp.int32, (_N, 1), 0)

    def slice_round(Tc, H, D):
        G = H // (2 * D)
        Tr = Tc.reshape(G, 2, D, C)
        na = jnp.maximum(Tr[:, 0], Tr[:, 1])
        nb = jnp.minimum(Tr[:, 0], Tr[:, 1])
        return jnp.concatenate([na[:, None], nb[:, None]], axis=1).reshape(H, C)

    def roll_round(Tc, idx, D):
        up = jnp.concatenate([Tc[D:], Tc[:D]], axis=0)
        dn = jnp.concatenate([Tc[-D:], Tc[:-D]], axis=0)
        is_a = (idx & D) == 0
        partner = jnp.where(is_a, up, dn)
        mn = jnp.minimum(Tc, partner)
        mx = jnp.maximum(Tc, partner)
        return jnp.where(is_a, mx, mn)

    cur = (sidx >> _pbit(1)) & 1
    T = jnp.where(cur == 1, -T, T)
    for st in range(1, 12):
        for j in range(st - 1, -1, -1):
            D = 1 << _pbit(j)
            if D >= 8:
                T = slice_round(T, _N, D)
            else:
                T = roll_round(T, sidx, D)
        nxt = ((sidx >> _pbit(st + 1)) & 1) if st < 11 else jnp.zeros_like(sidx)
        flip = cur ^ nxt
        T = jnp.where(flip == 1, -T, T)
        cur = nxt

    Tr = T.reshape(512, 2, 4, C)
    T2 = jnp.maximum(Tr[:, 0], Tr[:, 1]).reshape(_K, C)
    s2idx = lax.broadcasted_iota(jnp.int32, (_K, 1), 0)
    for j in range(10, -1, -1):
        if j >= 9:
            T2 = roll_round(T2, s2idx, 1 << (j - 9))
        elif j >= 1:
            T2 = slice_round(T2, _K, 1 << (j + 2))
        else:
            T2 = roll_round(T2, s2idx, 4)

    o_ref[...] = T2.reshape(512, 4, C)


def kernel(x):
    B, CH, N = x.shape
    R = B * CH
    xr = x.reshape(R, 8, 512).transpose(2, 1, 0).reshape(_N, R)
    C = 128
    out = pl.pallas_call(
        _sort_block,
        grid=(R // C,),
        in_specs=[pl.BlockSpec((_N, C), lambda g: (0, g))],
        out_specs=pl.BlockSpec((_N // 8, 4, C), lambda g: (0, 0, g)),
        out_shape=jax.ShapeDtypeStruct((_N // 8, 4, R), jnp.float32),
    )(xr)
    y = out.transpose(2, 1, 0).reshape(R, _K)
    return y.reshape(B, CH, _K)


# speedup vs baseline: 4.5038x; 1.2073x over previous
import jax
import jax.numpy as jnp
from jax import lax
from jax.experimental import pallas as pl

_N = 4096
_K = 2048


def _pbit(b):
    return b + 3 if b < 9 else b - 9


def _sort_block(x_ref, o_ref):
    T = x_ref[...]
    C = T.shape[1]
    sidx = lax.broadcasted_iota(jnp.int32, (_N, 1), 0)

    def slice_round(Tc, H, D):
        G = H // (2 * D)
        Tr = Tc.reshape(G, 2, D, C)
        na = jnp.maximum(Tr[:, 0], Tr[:, 1])
        nb = jnp.minimum(Tr[:, 0], Tr[:, 1])
        return jnp.concatenate([na[:, None], nb[:, None]], axis=1).reshape(H, C)

    def roll_round(Tc, idx, D):
        up = jnp.concatenate([Tc[D:], Tc[:D]], axis=0)
        dn = jnp.concatenate([Tc[-D:], Tc[:-D]], axis=0)
        is_a = (idx & D) == 0
        partner = jnp.where(is_a, up, dn)
        mn = jnp.minimum(Tc, partner)
        mx = jnp.maximum(Tc, partner)
        return jnp.where(is_a, mx, mn)

    for st in range(1, 9):
        for j in range(st - 1, -1, -1):
            Dj = 1 << (j + 3)
            A = _N >> (st + 4)
            Bm = 1 << (st - j - 1)
            Tr = T.reshape(A, 2, Bm, 2, Dj, C)
            d = Tr[:, 0]
            e = Tr[:, 1]
            nda = jnp.maximum(d[:, :, 0], d[:, :, 1])
            ndb = jnp.minimum(d[:, :, 0], d[:, :, 1])
            nea = jnp.minimum(e[:, :, 0], e[:, :, 1])
            neb = jnp.maximum(e[:, :, 0], e[:, :, 1])
            nd = jnp.concatenate([nda[:, :, None], ndb[:, :, None]], axis=2)
            ne = jnp.concatenate([nea[:, :, None], neb[:, :, None]], axis=2)
            T = jnp.concatenate([nd[:, None], ne[:, None]], axis=1)
            T = T.reshape(_N, C)

    cur = (sidx >> _pbit(9)) & 1
    T = jnp.where(cur == 1, -T, T)
    for st in range(9, 12):
        for j in range(st - 1, -1, -1):
            D = 1 << _pbit(j)
            if D >= 8:
                T = slice_round(T, _N, D)
            else:
                T = roll_round(T, sidx, D)
        nxt = ((sidx >> _pbit(st + 1)) & 1) if st < 11 else jnp.zeros_like(sidx)
        flip = cur ^ nxt
        T = jnp.where(flip == 1, -T, T)
        cur = nxt

    Tr = T.reshape(512, 2, 4, C)
    T2 = jnp.maximum(Tr[:, 0], Tr[:, 1]).reshape(_K, C)
    s2idx = lax.broadcasted_iota(jnp.int32, (_K, 1), 0)
    for j in range(10, -1, -1):
        if j >= 9:
            T2 = roll_round(T2, s2idx, 1 << (j - 9))
        elif j >= 1:
            T2 = slice_round(T2, _K, 1 << (j + 2))
        else:
            T2 = roll_round(T2, s2idx, 4)

    o_ref[...] = T2.reshape(512, 4, C)


def kernel(x):
    B, CH, N = x.shape
    R = B * CH
    xr = x.reshape(R, 8, 512).transpose(2, 1, 0).reshape(_N, R)
    C = 128
    out = pl.pallas_call(
        _sort_block,
        grid=(R // C,),
        in_specs=[pl.BlockSpec((_N, C), lambda g: (0, g))],
        out_specs=pl.BlockSpec((_N // 8, 4, C), lambda g: (0, 0, g)),
        out_shape=jax.ShapeDtypeStruct((_N // 8, 4, R), jnp.float32),
    )(xr)
    y = out.transpose(2, 1, 0).reshape(R, _K)
    return y.reshape(B, CH, _K)
